# baseline (speedup 1.0000x reference)
import functools

import jax
import jax.numpy as jnp
from jax import lax
from jax.experimental import pallas as pl
from jax.experimental.pallas import tpu as pltpu
from jax.experimental.pallas import tpu_sc as plsc

NC = 2
NS = 16
CHUNK = 128
NBUF = 4
RCH = 128
HALF = 64


def _ln_body(h_ref, g_ref, b_ref, o_ref):
    x = h_ref[...]
    mu = jnp.mean(x, axis=-1, keepdims=True)
    xc = x - mu
    var = jnp.mean(xc * xc, axis=-1, keepdims=True)
    o_ref[...] = xc * lax.rsqrt(var + 1e-5) * g_ref[...] + b_ref[...]


def _layernorm(h, gamma, beta, block_rows):
    n, d = h.shape
    return pl.pallas_call(
        _ln_body,
        grid=(n // block_rows,),
        in_specs=[
            pl.BlockSpec((block_rows, d), lambda i: (i, 0)),
            pl.BlockSpec((1, d), lambda i: (0, 0)),
            pl.BlockSpec((1, d), lambda i: (0, 0)),
        ],
        out_specs=pl.BlockSpec((block_rows, d), lambda i: (i, 0)),
        out_shape=jax.ShapeDtypeStruct((n, d), jnp.float32),
    )(h, gamma.reshape(1, d), beta.reshape(1, d))


def _make_sc_pass(h_pad, rpt, nch0, nch1, with_deg):
    n_chunks = max(nch0, nch1)
    mesh = plsc.VectorSubcoreMesh(core_axis_name="c", subcore_axis_name="s",
                                  num_cores=NC)
    out_type = jax.ShapeDtypeStruct((NC, h_pad, HALF), jnp.float32)
    if with_deg:
        out_type = (out_type,
                    jax.ShapeDtypeStruct((NC, NS, h_pad), jnp.float32))

    @functools.partial(
        pl.kernel,
        out_type=out_type,
        mesh=mesh,
        scratch_types=[
            pltpu.VMEM((n_chunks, CHUNK), jnp.int32),
            pltpu.VMEM((n_chunks, CHUNK), jnp.int32),
            pltpu.VMEM((NBUF, CHUNK, HALF), jnp.float32),
            pltpu.VMEM((h_pad,), jnp.float32),
            pltpu.VMEM_SHARED((h_pad, HALF), jnp.float32),
        ]
        + [pltpu.SemaphoreType.DMA] * (2 * NBUF),
        compiler_params=pltpu.CompilerParams(use_tc_tiling_on_sc=False,
                                             needs_layout_passes=False),
    )
    def sc_pass(hn2_hbm, src_hbm, dst_hbm, z_hbm, agg_out, *rest):
        if with_deg:
            deg_out = rest[0]
            rest = rest[1:]
        src_v, dst_v, rows_v, deg_v, agg_sp = rest[:5]
        sems = rest[5:]
        gsems = sems[0:NBUF]
        ssems = sems[NBUF:2 * NBUF]
        cid = lax.axis_index("c")
        sid = lax.axis_index("s")
        row0 = sid * rpt
        groups_d = jnp.where(cid == 0, nch0 // NBUF, nch1 // NBUF)
        row_off = jnp.where(cid == 0, sid * nch0, NS * nch0 + sid * nch1)

        pltpu.sync_copy(src_hbm.at[pl.ds(row_off, n_chunks)], src_v)
        pltpu.sync_copy(dst_hbm.at[pl.ds(row_off, n_chunks)], dst_v)

        pltpu.sync_copy(z_hbm, rows_v.at[0, pl.ds(0, RCH)])
        for t in range(rpt // RCH):
            pltpu.sync_copy(rows_v.at[0, pl.ds(0, RCH)],
                            agg_sp.at[pl.ds(row0 + t * RCH, RCH)])

        if with_deg:
            zeros16 = jnp.zeros((16,), jnp.float32)
            ones16 = jnp.full((16,), 1.0, jnp.float32)

            def zero_deg(i, _):
                deg_v[pl.ds(i * 16, 16)] = zeros16
                return 0
            lax.fori_loop(0, h_pad // 16, zero_deg, 0)

            def count(i, _):
                dvec = dst_v[i // (CHUNK // 16),
                             pl.ds((i % (CHUNK // 16)) * 16, 16)]
                plsc.addupdate_scatter(deg_v, [dvec], ones16)
                return 0
            lax.fori_loop(0, groups_d * NBUF * (CHUNK // 16), count, 0)
            pltpu.sync_copy(deg_v, deg_out.at[cid, sid])

        plsc.subcore_barrier()

        def start_gather(j, b):
            pltpu.async_copy(hn2_hbm.at[src_v.at[j]], rows_v.at[b], gsems[b])

        def wait_gather(j, b):
            pltpu.make_async_copy(
                hn2_hbm.at[src_v.at[j]], rows_v.at[b], gsems[b]).wait()

        def start_scatter(j, b):
            pltpu.async_copy(
                rows_v.at[b], agg_sp.at[dst_v.at[j]], ssems[b], add=True)

        def wait_scatter(j, b):
            pltpu.make_async_copy(
                rows_v.at[b], agg_sp.at[dst_v.at[j]], ssems[b]).wait()

        M = NBUF // 2
        for b in range(NBUF):
            start_gather(b, b)

        def pair(i, _):
            c0 = i * NBUF
            for m in range(M):
                wait_gather(c0 + m, m)
                start_scatter(c0 + m, m)
            for m in range(M):
                wait_gather(c0 + M + m, M + m)
                start_scatter(c0 + M + m, M + m)
            for m in range(M):
                wait_scatter(c0 + m, m)
                start_gather(c0 + NBUF + m, m)
            for m in range(M):
                wait_scatter(c0 + M + m, M + m)
                start_gather(c0 + NBUF + M + m, M + m)
            return 0
        lax.fori_loop(0, groups_d - 1, pair, 0)

        cL = (groups_d - 1) * NBUF
        for m in range(M):
            wait_gather(cL + m, m)
            start_scatter(cL + m, m)
        for m in range(M):
            wait_gather(cL + M + m, M + m)
            start_scatter(cL + M + m, M + m)
        for b in range(NBUF):
            wait_scatter(cL + b, b)

        plsc.subcore_barrier()

        noc = rpt // RCH
        for t in range(noc):
            s = t % NBUF
            if t >= NBUF:
                pltpu.make_async_copy(
                    rows_v.at[s, pl.ds(0, RCH)],
                    agg_out.at[cid, pl.ds(row0 + (t - NBUF) * RCH, RCH)],
                    ssems[s]).wait()
            pltpu.sync_copy(agg_sp.at[pl.ds(row0 + t * RCH, RCH)],
                            rows_v.at[s, pl.ds(0, RCH)])
            pltpu.async_copy(
                rows_v.at[s, pl.ds(0, RCH)],
                agg_out.at[cid, pl.ds(row0 + t * RCH, RCH)], ssems[s])
        for t in range(max(noc - NBUF, 0), noc):
            s = t % NBUF
            pltpu.make_async_copy(
                rows_v.at[s, pl.ds(0, RCH)],
                agg_out.at[cid, pl.ds(row0 + t * RCH, RCH)],
                ssems[s]).wait()

    return sc_pass


def _final_body(hn_ref, a0_ref, a1_ref, deg_ref, ws_ref, wn_ref, b_ref,
                o_ref):
    hn = hn_ref[...]
    agg = jnp.concatenate([jnp.sum(a0_ref[...], axis=0),
                           jnp.sum(a1_ref[...], axis=0)], axis=-1)
    deg = jnp.maximum(jnp.sum(deg_ref[...], axis=(0, 1)), 1.0)
    hng = agg / deg[:, None]
    dn = (((1,), (1,)), ((), ()))
    o_ref[...] = (
        lax.dot_general(hn, ws_ref[...], dn, preferred_element_type=jnp.float32)
        + lax.dot_general(hng, wn_ref[...], dn, preferred_element_type=jnp.float32)
        + b_ref[...]
    )


def _final(hn, agg0, agg1, deg_parts, W_self, W_neigh, bias, block_rows):
    n, d = hn.shape
    return pl.pallas_call(
        _final_body,
        grid=(-(-n // block_rows),),
        in_specs=[
            pl.BlockSpec((block_rows, d), lambda i: (i, 0)),
            pl.BlockSpec((NC, block_rows, HALF), lambda i: (0, i, 0)),
            pl.BlockSpec((NC, block_rows, HALF), lambda i: (0, i, 0)),
            pl.BlockSpec((NC, NS, block_rows), lambda i: (0, 0, i)),
            pl.BlockSpec((d, d), lambda i: (0, 0)),
            pl.BlockSpec((d, d), lambda i: (0, 0)),
            pl.BlockSpec((1, d), lambda i: (0, 0)),
        ],
        out_specs=pl.BlockSpec((block_rows, d), lambda i: (i, 0)),
        out_shape=jax.ShapeDtypeStruct((n, d), jnp.float32),
    )(hn, agg0, agg1, deg_parts, W_self, W_neigh, bias.reshape(1, d))


def kernel(h, edge_index, ln_gamma, ln_beta, W_self, W_neigh, bias):
    n, d = h.shape
    e = edge_index.shape[1]

    rpt = -(-(n + 1) // (NS * RCH)) * RCH
    h_pad = NS * rpt
    nch = -(-(-(-e // (NC * NS * CHUNK))) // NBUF) * NBUF
    nch0 = max((2 * nch // 4) // NBUF * NBUF, NBUF)
    nch1 = 2 * nch - nch0
    e_pad = NS * (nch0 + nch1) * CHUNK
    pad = e_pad - e

    src = jnp.concatenate([edge_index[0], jnp.zeros((pad,), jnp.int32)])
    src2 = jnp.stack([2 * src, 2 * src + 1]).reshape(2, -1, CHUNK)
    dst = jnp.concatenate(
        [edge_index[1], jnp.full((pad,), n, jnp.int32)]).reshape(-1, CHUNK)
    zrows = jnp.zeros((RCH, HALF), jnp.float32)

    hn = _layernorm(h, ln_gamma, ln_beta, block_rows=1000)
    hn2 = hn.reshape(2 * n, HALF)
    agg0, deg_parts = _make_sc_pass(h_pad, rpt, nch0, nch1, True)(
        hn2, src2[0], dst, zrows)
    agg1 = _make_sc_pass(h_pad, rpt, nch0, nch1, False)(
        hn2, src2[1], dst, zrows)
    return _final(hn, agg0, agg1, deg_parts, W_self, W_neigh, bias,
                  block_rows=1024)


# speedup vs baseline: 1.0520x; 1.0520x over previous
import functools

import jax
import jax.numpy as jnp
from jax import lax
from jax.experimental import pallas as pl
from jax.experimental.pallas import tpu as pltpu
from jax.experimental.pallas import tpu_sc as plsc

NC = 2
NS = 16
CHUNK = 128
NBUF = 4
RCH = 128
HALF = 64


def _ln_body(h_ref, g_ref, b_ref, o_ref):
    x = h_ref[...]
    mu = jnp.mean(x, axis=-1, keepdims=True)
    xc = x - mu
    var = jnp.mean(xc * xc, axis=-1, keepdims=True)
    o_ref[...] = xc * lax.rsqrt(var + 1e-5) * g_ref[...] + b_ref[...]


def _layernorm(h, gamma, beta, block_rows):
    n, d = h.shape
    return pl.pallas_call(
        _ln_body,
        grid=(n // block_rows,),
        in_specs=[
            pl.BlockSpec((block_rows, d), lambda i: (i, 0)),
            pl.BlockSpec((1, d), lambda i: (0, 0)),
            pl.BlockSpec((1, d), lambda i: (0, 0)),
        ],
        out_specs=pl.BlockSpec((block_rows, d), lambda i: (i, 0)),
        out_shape=jax.ShapeDtypeStruct((n, d), jnp.float32),
    )(h, gamma.reshape(1, d), beta.reshape(1, d))


def _make_sc_pass(h_pad, rpt, n_chunks, with_deg):
    groups = n_chunks // NBUF
    mesh = plsc.VectorSubcoreMesh(core_axis_name="c", subcore_axis_name="s",
                                  num_cores=NC)
    out_type = jax.ShapeDtypeStruct((NC, h_pad, HALF), jnp.float32)
    if with_deg:
        out_type = (out_type,
                    jax.ShapeDtypeStruct((NC, NS, h_pad), jnp.float32))

    @functools.partial(
        pl.kernel,
        out_type=out_type,
        mesh=mesh,
        scratch_types=[
            pltpu.VMEM((n_chunks, CHUNK), jnp.int32),
            pltpu.VMEM((n_chunks, CHUNK), jnp.int32),
            pltpu.VMEM((NBUF, CHUNK, HALF), jnp.float32),
            pltpu.VMEM((h_pad,), jnp.float32),
            pltpu.VMEM_SHARED((h_pad, HALF), jnp.float32),
        ]
        + [pltpu.SemaphoreType.DMA] * (2 * NBUF),
        compiler_params=pltpu.CompilerParams(use_tc_tiling_on_sc=False,
                                             needs_layout_passes=False),
    )
    def sc_pass(hn2_hbm, src_hbm, dst_hbm, z_hbm, agg_out, *rest):
        if with_deg:
            deg_out = rest[0]
            rest = rest[1:]
        src_v, dst_v, rows_v, deg_v, agg_sp = rest[:5]
        sems = rest[5:]
        gsems = sems[0:NBUF]
        ssems = sems[NBUF:2 * NBUF]
        cid = lax.axis_index("c")
        sid = lax.axis_index("s")
        row0 = sid * rpt

        pltpu.sync_copy(src_hbm.at[cid, sid], src_v)
        pltpu.sync_copy(dst_hbm.at[cid, sid], dst_v)

        pltpu.sync_copy(z_hbm, rows_v.at[0, pl.ds(0, RCH)])
        for t in range(rpt // RCH):
            pltpu.sync_copy(rows_v.at[0, pl.ds(0, RCH)],
                            agg_sp.at[pl.ds(row0 + t * RCH, RCH)])

        if with_deg:
            zeros16 = jnp.zeros((16,), jnp.float32)
            ones16 = jnp.full((16,), 1.0, jnp.float32)

            def zero_deg(i, _):
                deg_v[pl.ds(i * 16, 16)] = zeros16
                return 0
            lax.fori_loop(0, h_pad // 16, zero_deg, 0)

            def count(i, _):
                dvec = dst_v[i // (CHUNK // 16), pl.ds((i % (CHUNK // 16)) * 16, 16)]
                plsc.addupdate_scatter(deg_v, [dvec], ones16)
                return 0
            lax.fori_loop(0, n_chunks * (CHUNK // 16), count, 0)
            pltpu.sync_copy(deg_v, deg_out.at[cid, sid])

        plsc.subcore_barrier()

        def start_gather(j, b):
            pltpu.async_copy(hn2_hbm.at[src_v.at[j]], rows_v.at[b], gsems[b])

        def wait_gather(j, b):
            pltpu.make_async_copy(
                hn2_hbm.at[src_v.at[j]], rows_v.at[b], gsems[b]).wait()

        def start_scatter(j, b):
            pltpu.async_copy(
                rows_v.at[b], agg_sp.at[dst_v.at[j]], ssems[b], add=True)

        def wait_scatter(j, b):
            pltpu.make_async_copy(
                rows_v.at[b], agg_sp.at[dst_v.at[j]], ssems[b]).wait()

        M = NBUF // 2
        for b in range(NBUF):
            start_gather(b, b)

        def pair(i, _):
            c0 = i * NBUF
            for m in range(M):
                wait_gather(c0 + m, m)
                start_scatter(c0 + m, m)
            for m in range(M):
                wait_gather(c0 + M + m, M + m)
                start_scatter(c0 + M + m, M + m)
            for m in range(M):
                wait_scatter(c0 + m, m)
                start_gather(c0 + NBUF + m, m)
            for m in range(M):
                wait_scatter(c0 + M + m, M + m)
                start_gather(c0 + NBUF + M + m, M + m)
            return 0
        lax.fori_loop(0, groups - 1, pair, 0)

        cL = (groups - 1) * NBUF
        for m in range(M):
            wait_gather(cL + m, m)
            start_scatter(cL + m, m)
        for m in range(M):
            wait_gather(cL + M + m, M + m)
            start_scatter(cL + M + m, M + m)
        for b in range(NBUF):
            wait_scatter(cL + b, b)

        plsc.subcore_barrier()

        noc = rpt // RCH
        for t in range(noc):
            s = t % NBUF
            if t >= NBUF:
                pltpu.make_async_copy(
                    rows_v.at[s, pl.ds(0, RCH)],
                    agg_out.at[cid, pl.ds(row0 + (t - NBUF) * RCH, RCH)],
                    ssems[s]).wait()
            pltpu.sync_copy(agg_sp.at[pl.ds(row0 + t * RCH, RCH)],
                            rows_v.at[s, pl.ds(0, RCH)])
            pltpu.async_copy(
                rows_v.at[s, pl.ds(0, RCH)],
                agg_out.at[cid, pl.ds(row0 + t * RCH, RCH)], ssems[s])
        for t in range(max(noc - NBUF, 0), noc):
            s = t % NBUF
            pltpu.make_async_copy(
                rows_v.at[s, pl.ds(0, RCH)],
                agg_out.at[cid, pl.ds(row0 + t * RCH, RCH)],
                ssems[s]).wait()

    return sc_pass


def _final_body(hn_ref, a0_ref, a1_ref, deg_ref, ws_ref, wn_ref, b_ref,
                o_ref):
    hn = hn_ref[...]
    agg = jnp.concatenate([jnp.sum(a0_ref[...], axis=0),
                           jnp.sum(a1_ref[...], axis=0)], axis=-1)
    deg = jnp.maximum(jnp.sum(deg_ref[...], axis=(0, 1)), 1.0)
    hng = agg / deg[:, None]
    dn = (((1,), (1,)), ((), ()))
    o_ref[...] = (
        lax.dot_general(hn, ws_ref[...], dn, preferred_element_type=jnp.float32)
        + lax.dot_general(hng, wn_ref[...], dn, preferred_element_type=jnp.float32)
        + b_ref[...]
    )


def _final(hn, agg0, agg1, deg_parts, W_self, W_neigh, bias, block_rows):
    n, d = hn.shape
    return pl.pallas_call(
        _final_body,
        grid=(-(-n // block_rows),),
        in_specs=[
            pl.BlockSpec((block_rows, d), lambda i: (i, 0)),
            pl.BlockSpec((NC, block_rows, HALF), lambda i: (0, i, 0)),
            pl.BlockSpec((NC, block_rows, HALF), lambda i: (0, i, 0)),
            pl.BlockSpec((NC, NS, block_rows), lambda i: (0, 0, i)),
            pl.BlockSpec((d, d), lambda i: (0, 0)),
            pl.BlockSpec((d, d), lambda i: (0, 0)),
            pl.BlockSpec((1, d), lambda i: (0, 0)),
        ],
        out_specs=pl.BlockSpec((block_rows, d), lambda i: (i, 0)),
        out_shape=jax.ShapeDtypeStruct((n, d), jnp.float32),
    )(hn, agg0, agg1, deg_parts, W_self, W_neigh, bias.reshape(1, d))


def kernel(h, edge_index, ln_gamma, ln_beta, W_self, W_neigh, bias):
    n, d = h.shape
    e = edge_index.shape[1]

    rpt = -(-(n + 1) // (NS * RCH)) * RCH
    h_pad = NS * rpt
    n_chunks = -(-(-(-e // (NC * NS * CHUNK))) // NBUF) * NBUF
    e_pad = NC * NS * n_chunks * CHUNK
    pad = e_pad - e

    src = jnp.concatenate([edge_index[0], jnp.zeros((pad,), jnp.int32)])
    src2 = jnp.stack([2 * src, 2 * src + 1]).reshape(
        2, NC, NS, n_chunks, CHUNK)
    dst = jnp.concatenate(
        [edge_index[1], jnp.full((pad,), n, jnp.int32)]).reshape(
            NC, NS, n_chunks, CHUNK)
    zrows = jnp.zeros((RCH, HALF), jnp.float32)

    hn = _layernorm(h, ln_gamma, ln_beta, block_rows=1000)
    hn2 = hn.reshape(2 * n, HALF)
    agg0, deg_parts = _make_sc_pass(h_pad, rpt, n_chunks, True)(
        hn2, src2[0], dst, zrows)
    agg1 = _make_sc_pass(h_pad, rpt, n_chunks, False)(
        hn2, src2[1], dst, zrows)
    return _final(hn, agg0, agg1, deg_parts, W_self, W_neigh, bias,
                  block_rows=1024)
